# baseline (speedup 1.0000x reference)
import jax
import jax.numpy as jnp
from jax import lax
from jax.experimental import pallas as pl
from jax.experimental.pallas import tpu as pltpu
from jax.experimental.pallas import tpu_sc as plsc

B = 16384
D = 64
HIDDEN = 128
NC = 2
NS = 16
NW = NC * NS
BPW = B // NW
CH = 128
NCH = BPW // CH


def _gather_body(u_idx_hbm, m_idx_hbm, u_emb_hbm, m_emb_hbm,
                 u_bias_hbm, m_bias_hbm,
                 u_out, m_out, ub_out, mb_out,
                 uidx_v, midx_v, u_rows, m_rows, ub_v, mb_v,
                 sem_u, sem_m, sem_b):
    wid = lax.axis_index("s") * NC + lax.axis_index("c")
    base = wid * BPW
    row0 = wid * NCH
    pltpu.sync_copy(u_idx_hbm.at[pl.ds(row0, NCH)], uidx_v)
    pltpu.sync_copy(m_idx_hbm.at[pl.ds(row0, NCH)], midx_v)
    copies = []
    for j in range(NCH):
        copies.append(pltpu.async_copy(
            u_emb_hbm.at[uidx_v.at[j]], u_rows.at[pl.ds(j * CH, CH)], sem_u))
        copies.append(pltpu.async_copy(
            m_emb_hbm.at[midx_v.at[j]], m_rows.at[pl.ds(j * CH, CH)], sem_m))
        copies.append(pltpu.async_copy(
            u_bias_hbm.at[uidx_v.at[j]], ub_v.at[j], sem_b))
        copies.append(pltpu.async_copy(
            m_bias_hbm.at[midx_v.at[j]], mb_v.at[j], sem_b))
    for c in copies:
        c.wait()
    pltpu.sync_copy(u_rows, u_out.at[pl.ds(base, BPW)])
    pltpu.sync_copy(m_rows, m_out.at[pl.ds(base, BPW)])
    pltpu.sync_copy(ub_v, ub_out.at[pl.ds(row0, NCH)])
    pltpu.sync_copy(mb_v, mb_out.at[pl.ds(row0, NCH)])


_gather = pl.kernel(
    _gather_body,
    mesh=plsc.VectorSubcoreMesh(core_axis_name="c", subcore_axis_name="s"),
    out_type=[
        jax.ShapeDtypeStruct((B, D), jnp.float32),
        jax.ShapeDtypeStruct((B, D), jnp.float32),
        jax.ShapeDtypeStruct((B // CH, CH), jnp.float32),
        jax.ShapeDtypeStruct((B // CH, CH), jnp.float32),
    ],
    scratch_types=[
        pltpu.VMEM((NCH, CH), jnp.int32),
        pltpu.VMEM((NCH, CH), jnp.int32),
        pltpu.VMEM((BPW, D), jnp.float32),
        pltpu.VMEM((BPW, D), jnp.float32),
        pltpu.VMEM((NCH, CH), jnp.float32),
        pltpu.VMEM((NCH, CH), jnp.float32),
        pltpu.SemaphoreType.DMA,
        pltpu.SemaphoreType.DMA,
        pltpu.SemaphoreType.DMA,
    ],
    compiler_params=pltpu.CompilerParams(use_tc_tiling_on_sc=False),
)


BT = 1024
GRID = B // BT
PR = BT // CH


def _mlp_body(u_ref, m_ref, ub_ref, mb_ref, w1_ref, b1_ref, w2_ref, b2_ref,
              o_ref):
    u = u_ref[...]
    m = m_ref[...]
    w1 = w1_ref[...]
    P = lax.Precision.HIGHEST
    acc = jnp.dot(u, w1[0:D, :], preferred_element_type=jnp.float32,
                  precision=P)
    acc = acc + jnp.dot(m, w1[D:2 * D, :], preferred_element_type=jnp.float32,
                        precision=P)
    acc = acc + jnp.dot(u * m, w1[2 * D:3 * D, :],
                        preferred_element_type=jnp.float32, precision=P)
    h = jnp.maximum(acc + b1_ref[...], 0.0)
    o = jnp.dot(h, w2_ref[...], preferred_element_type=jnp.float32,
                precision=P)
    bb = ub_ref[...] + mb_ref[...]
    row_ids = lax.broadcasted_iota(jnp.int32, (BT, PR), 0) // CH
    col_ids = lax.broadcasted_iota(jnp.int32, (BT, PR), 1)
    rowsel = (row_ids == col_ids).astype(jnp.float32)
    expanded = jnp.dot(rowsel, bb, preferred_element_type=jnp.float32,
                       precision=P)
    lane = lax.broadcasted_iota(jnp.int32, (BT, CH), 1)
    bpos = lax.broadcasted_iota(jnp.int32, (BT, CH), 0) % CH
    bias_col = jnp.sum(jnp.where(lane == bpos, expanded, 0.0), axis=1,
                       keepdims=True)
    o_ref[...] = jax.nn.sigmoid(o + b2_ref[0, 0] + bias_col)


_mlp = pl.pallas_call(
    _mlp_body,
    grid=(GRID,),
    in_specs=[
        pl.BlockSpec((BT, D), lambda i: (i, 0)),
        pl.BlockSpec((BT, D), lambda i: (i, 0)),
        pl.BlockSpec((PR, CH), lambda i: (i, 0)),
        pl.BlockSpec((PR, CH), lambda i: (i, 0)),
        pl.BlockSpec((3 * D, HIDDEN), lambda i: (0, 0)),
        pl.BlockSpec((1, HIDDEN), lambda i: (0, 0)),
        pl.BlockSpec((HIDDEN, 1), lambda i: (0, 0)),
        pl.BlockSpec((1, 1), lambda i: (0, 0)),
    ],
    out_specs=pl.BlockSpec((BT, 1), lambda i: (i, 0)),
    out_shape=jax.ShapeDtypeStruct((B, 1), jnp.float32),
)


def kernel(u_idx, m_idx, u_emb, m_emb, u_bias, m_bias, W1, b1, W2, b2):
    u_idx2 = u_idx.astype(jnp.int32).reshape(B // CH, CH)
    m_idx2 = m_idx.astype(jnp.int32).reshape(B // CH, CH)
    u_g, m_g, ub_g, mb_g = _gather(u_idx2, m_idx2, u_emb, m_emb,
                                   u_bias.reshape(-1), m_bias.reshape(-1))
    out = _mlp(u_g, m_g, ub_g, mb_g, W1, b1.reshape(1, HIDDEN), W2,
               b2.reshape(1, 1))
    return out.reshape(B)


# speedup vs baseline: 1.0228x; 1.0228x over previous
import jax
import jax.numpy as jnp
from jax import lax
from jax.experimental import pallas as pl
from jax.experimental.pallas import tpu as pltpu
from jax.experimental.pallas import tpu_sc as plsc

B = 16384
D = 64
HIDDEN = 128
NC = 2
NS = 16
NW = NC * NS
BPW = B // NW
CH = 128
NCH = BPW // CH


def _gather_body(u_idx_hbm, m_idx_hbm, u_emb_hbm, m_emb_hbm,
                 u_bias_hbm, m_bias_hbm,
                 x_out, ub_out, mb_out,
                 uidx_v, midx_v, u_rows, m_rows, ub_v, mb_v,
                 sem_u, sem_m, sem_b):
    wid = lax.axis_index("s") * NC + lax.axis_index("c")
    base = wid * BPW
    prow0 = wid * NCH
    pltpu.sync_copy(u_idx_hbm.at[pl.ds(base, BPW)], uidx_v)
    pltpu.sync_copy(m_idx_hbm.at[pl.ds(base, BPW)], midx_v)
    copies = []
    for j in range(NCH):
        sl = pl.ds(j * CH, CH)
        copies.append(pltpu.async_copy(
            u_emb_hbm.at[uidx_v.at[sl]], u_rows.at[pl.ds(j * CH, CH)], sem_u))
        copies.append(pltpu.async_copy(
            m_emb_hbm.at[midx_v.at[sl]], m_rows.at[pl.ds(j * CH, CH)], sem_m))
        copies.append(pltpu.async_copy(
            u_bias_hbm.at[uidx_v.at[sl]], ub_v.at[j], sem_b))
        copies.append(pltpu.async_copy(
            m_bias_hbm.at[midx_v.at[sl]], mb_v.at[j], sem_b))
    for c in copies:
        c.wait()
    pltpu.sync_copy(u_rows, x_out.at[pl.ds(base, BPW), pl.ds(0, D)])
    pltpu.sync_copy(m_rows, x_out.at[pl.ds(base, BPW), pl.ds(D, D)])
    pltpu.sync_copy(ub_v, ub_out.at[pl.ds(prow0, NCH)])
    pltpu.sync_copy(mb_v, mb_out.at[pl.ds(prow0, NCH)])


_gather = pl.kernel(
    _gather_body,
    mesh=plsc.VectorSubcoreMesh(core_axis_name="c", subcore_axis_name="s"),
    out_type=[
        jax.ShapeDtypeStruct((B, 2 * D), jnp.float32),
        jax.ShapeDtypeStruct((B // CH, CH), jnp.float32),
        jax.ShapeDtypeStruct((B // CH, CH), jnp.float32),
    ],
    scratch_types=[
        pltpu.VMEM((BPW,), jnp.int32),
        pltpu.VMEM((BPW,), jnp.int32),
        pltpu.VMEM((BPW, D), jnp.float32),
        pltpu.VMEM((BPW, D), jnp.float32),
        pltpu.VMEM((NCH, CH), jnp.float32),
        pltpu.VMEM((NCH, CH), jnp.float32),
        pltpu.SemaphoreType.DMA,
        pltpu.SemaphoreType.DMA,
        pltpu.SemaphoreType.DMA,
    ],
    compiler_params=pltpu.CompilerParams(use_tc_tiling_on_sc=False),
)


BT = 1024
GRID = B // BT
PR = BT // CH


def _mlp_body(x_ref, ub_ref, mb_ref, w1_ref, b1_ref, w2_ref, b2_ref, o_ref):
    x = x_ref[...]
    w1 = w1_ref[...]
    P = lax.Precision.HIGHEST
    acc = jnp.dot(x, w1[0:2 * D, :], preferred_element_type=jnp.float32,
                  precision=P)
    inter = x[:, 0:D] * x[:, D:2 * D]
    acc = acc + jnp.dot(inter, w1[2 * D:3 * D, :],
                        preferred_element_type=jnp.float32, precision=P)
    h = jnp.maximum(acc + b1_ref[...], 0.0)
    o = jnp.dot(h, w2_ref[...], preferred_element_type=jnp.float32,
                precision=P)
    bb = ub_ref[...] + mb_ref[...]
    row_ids = lax.broadcasted_iota(jnp.int32, (BT, PR), 0) // CH
    col_ids = lax.broadcasted_iota(jnp.int32, (BT, PR), 1)
    rowsel = (row_ids == col_ids).astype(jnp.float32)
    expanded = jnp.dot(rowsel, bb, preferred_element_type=jnp.float32,
                       precision=P)
    lane = lax.broadcasted_iota(jnp.int32, (BT, CH), 1)
    bpos = lax.broadcasted_iota(jnp.int32, (BT, CH), 0) % CH
    bias_col = jnp.sum(jnp.where(lane == bpos, expanded, 0.0), axis=1,
                       keepdims=True)
    o_ref[...] = jax.nn.sigmoid(o + b2_ref[0, 0] + bias_col)


_mlp = pl.pallas_call(
    _mlp_body,
    grid=(GRID,),
    in_specs=[
        pl.BlockSpec((BT, 2 * D), lambda i: (i, 0)),
        pl.BlockSpec((PR, CH), lambda i: (i, 0)),
        pl.BlockSpec((PR, CH), lambda i: (i, 0)),
        pl.BlockSpec((3 * D, HIDDEN), lambda i: (0, 0)),
        pl.BlockSpec((1, HIDDEN), lambda i: (0, 0)),
        pl.BlockSpec((HIDDEN, 1), lambda i: (0, 0)),
        pl.BlockSpec((1, 1), lambda i: (0, 0)),
    ],
    out_specs=pl.BlockSpec((BT, 1), lambda i: (i, 0)),
    out_shape=jax.ShapeDtypeStruct((B, 1), jnp.float32),
)


def kernel(u_idx, m_idx, u_emb, m_emb, u_bias, m_bias, W1, b1, W2, b2):
    x_g, ub_g, mb_g = _gather(u_idx.astype(jnp.int32), m_idx.astype(jnp.int32),
                              u_emb, m_emb, u_bias.reshape(-1),
                              m_bias.reshape(-1))
    out = _mlp(x_g, ub_g, mb_g, W1, b1.reshape(1, HIDDEN), W2,
               b2.reshape(1, 1))
    return out.reshape(B)
